# baseline (speedup 1.0000x reference)
import functools

import jax
import jax.numpy as jnp
from jax.experimental import pallas as pl
from jax.experimental.pallas import tpu as pltpu

B = 4096
K = 8
N = 30
IN_C = 128
F = 64
C = 5
BB = 256


def _fused_kernel(real_ref, graph_ref, w1_ref, b1_ref, w2_ref, b2_ref,
                  wlin_ref, blin_ref, wheadt_ref, bhead_ref,
                  out1_ref, out2_ref, acc_ref):
    k = pl.program_id(1)
    g = graph_ref[...].reshape(N * N, BB)

    @pl.when(k == 0)
    def _init():
        acc_ref[...] = g

    @pl.when(k > 0)
    def _accum():
        acc_ref[...] += g

    @pl.when(k == K - 1)
    def _compute():
        adjT = acc_ref[...] * (1.0 / K)
        adj = adjT.T.reshape(BB, N, N)

        r = real_ref[...]
        h = jax.lax.dot_general(
            r, w1_ref[...],
            dimension_numbers=(((2,), (0,)), ((), ())),
            preferred_element_type=jnp.float32)
        h = jnp.transpose(h, (1, 0, 2))

        x = jax.lax.dot_general(
            adj, h,
            dimension_numbers=(((2,), (1,)), ((0,), (0,))),
            preferred_element_type=jnp.float32)
        x = jnp.maximum(x + b1_ref[...].reshape(1, 1, F), 0.0)

        h2 = jax.lax.dot_general(
            x, w2_ref[...],
            dimension_numbers=(((2,), (0,)), ((), ())),
            preferred_element_type=jnp.float32)
        x2 = jax.lax.dot_general(
            adj, h2,
            dimension_numbers=(((2,), (1,)), ((0,), (0,))),
            preferred_element_type=jnp.float32)
        x2 = jnp.maximum(x2 + b2_ref[...].reshape(1, 1, F), 0.0)

        wlin = wlin_ref[...].reshape(1, 1, F)
        blin = blin_ref[0, 0]
        wheadt = wheadt_ref[...]
        bhead = bhead_ref[...]

        def head(xk, out_ref):
            xl = jnp.sum(xk * wlin, axis=2) + blin
            xr = jnp.maximum(xl, 0.0)
            out = jax.lax.dot_general(
                xr, wheadt,
                dimension_numbers=(((1,), (0,)), ((), ())),
                preferred_element_type=jnp.float32)
            out_ref[...] = out + bhead

        head(x, out1_ref)
        head(x2, out2_ref)


@functools.partial(jax.jit, static_argnames=())
def _run(real, graph, W1, b1, W2, b2, Wlin, blin, Whead, bhead):
    gT = jnp.transpose(graph, (2, 3, 1, 0)).reshape(N * N, K, 1, B)
    rT = jnp.transpose(real, (1, 0, 2))
    grid = (B // BB, K)
    out1, out2 = pl.pallas_call(
        _fused_kernel,
        grid=grid,
        in_specs=[
            pl.BlockSpec((N, BB, IN_C), lambda i, k: (0, i, 0)),
            pl.BlockSpec((N * N, 1, 1, BB), lambda i, k: (0, k, 0, i)),
            pl.BlockSpec((IN_C, F), lambda i, k: (0, 0)),
            pl.BlockSpec((1, F), lambda i, k: (0, 0)),
            pl.BlockSpec((F, F), lambda i, k: (0, 0)),
            pl.BlockSpec((1, F), lambda i, k: (0, 0)),
            pl.BlockSpec((1, F), lambda i, k: (0, 0)),
            pl.BlockSpec((1, 1), lambda i, k: (0, 0)),
            pl.BlockSpec((N, C), lambda i, k: (0, 0)),
            pl.BlockSpec((1, C), lambda i, k: (0, 0)),
        ],
        out_specs=[
            pl.BlockSpec((BB, C), lambda i, k: (i, 0)),
            pl.BlockSpec((BB, C), lambda i, k: (i, 0)),
        ],
        out_shape=[
            jax.ShapeDtypeStruct((B, C), jnp.float32),
            jax.ShapeDtypeStruct((B, C), jnp.float32),
        ],
        scratch_shapes=[pltpu.VMEM((N * N, BB), jnp.float32)],
    )(rT, gT, W1, b1.reshape(1, F), W2, b2.reshape(1, F),
      Wlin.reshape(1, F), blin.reshape(1, 1), Whead.T, bhead.reshape(1, C))
    return out1, out2


def kernel(real, imag, graph, W1, b1, W2, b2, Wlin, blin, Whead, bhead, layer):
    del imag
    out1, out2 = _run(real, graph, W1, b1, W2, b2, Wlin, blin, Whead, bhead)
    return jnp.where(layer > 1, out2, out1)


# speedup vs baseline: 1.0668x; 1.0668x over previous
import functools

import jax
import jax.numpy as jnp
from jax.experimental import pallas as pl
from jax.experimental.pallas import tpu as pltpu

B = 4096
K = 8
N = 30
IN_C = 128
F = 64
C = 5
BB = 128


RR = 128
NNP = 1024


def _adj_kernel(g_ref, out_ref, acc_ref):
    k = pl.program_id(1)
    g = g_ref[...].reshape(RR, B)

    @pl.when(k == 0)
    def _init():
        acc_ref[...] = g

    @pl.when(k > 0)
    def _accum():
        acc_ref[...] += g

    @pl.when(k == K - 1)
    def _emit():
        out_ref[...] = acc_ref[...].T


def _net_kernel(adj_ref, real_ref, w1_ref, b1_ref, w2_ref, b2_ref,
                wlin_ref, blin_ref, wheadt_ref, bhead_ref,
                out1_ref, out2_ref):
    adj = adj_ref[:, : N * N].reshape(BB, N, N)

    r = real_ref[...]
    h = jax.lax.dot_general(
        r, w1_ref[...],
        dimension_numbers=(((2,), (0,)), ((), ())),
        preferred_element_type=jnp.float32)
    h = jnp.transpose(h, (1, 0, 2))

    x = jax.lax.dot_general(
        adj, h,
        dimension_numbers=(((2,), (1,)), ((0,), (0,))),
        preferred_element_type=jnp.float32)
    x = jnp.maximum(x + b1_ref[...].reshape(1, 1, F), 0.0)

    h2 = jax.lax.dot_general(
        x, w2_ref[...],
        dimension_numbers=(((2,), (0,)), ((), ())),
        preferred_element_type=jnp.float32)
    x2 = jax.lax.dot_general(
        adj, h2,
        dimension_numbers=(((2,), (1,)), ((0,), (0,))),
        preferred_element_type=jnp.float32)
    x2 = jnp.maximum(x2 + b2_ref[...].reshape(1, 1, F), 0.0)

    wlin = wlin_ref[...].reshape(1, 1, F)
    blin = blin_ref[0, 0]
    wheadt = wheadt_ref[...]
    bhead = bhead_ref[...]

    def head(xk, out_ref):
        xl = jnp.sum(xk * wlin, axis=2) + blin
        xr = jnp.maximum(xl, 0.0)
        out = jax.lax.dot_general(
            xr, wheadt,
            dimension_numbers=(((1,), (0,)), ((), ())),
            preferred_element_type=jnp.float32)
        out_ref[...] = out + bhead

    head(x, out1_ref)
    head(x2, out2_ref)


@functools.partial(jax.jit, static_argnames=())
def _run(real, graph, W1, b1, W2, b2, Wlin, blin, Whead, bhead):
    gT = jnp.transpose(graph, (2, 3, 1, 0)).reshape(N * N, K, 1, B)
    rT = jnp.transpose(real, (1, 0, 2))

    adjsum = pl.pallas_call(
        _adj_kernel,
        grid=(NNP // RR, K),
        in_specs=[pl.BlockSpec((RR, 1, 1, B), lambda r, k: (r, k, 0, 0))],
        out_specs=pl.BlockSpec((B, RR), lambda r, k: (0, r)),
        out_shape=jax.ShapeDtypeStruct((B, NNP), jnp.float32),
        scratch_shapes=[pltpu.VMEM((RR, B), jnp.float32)],
    )(gT)

    scale = jnp.float32(1.0 / K)
    out1, out2 = pl.pallas_call(
        _net_kernel,
        grid=(B // BB,),
        in_specs=[
            pl.BlockSpec((BB, NNP), lambda i: (i, 0)),
            pl.BlockSpec((N, BB, IN_C), lambda i: (0, i, 0)),
            pl.BlockSpec((IN_C, F), lambda i: (0, 0)),
            pl.BlockSpec((1, F), lambda i: (0, 0)),
            pl.BlockSpec((F, F), lambda i: (0, 0)),
            pl.BlockSpec((1, F), lambda i: (0, 0)),
            pl.BlockSpec((1, F), lambda i: (0, 0)),
            pl.BlockSpec((1, 1), lambda i: (0, 0)),
            pl.BlockSpec((N, C), lambda i: (0, 0)),
            pl.BlockSpec((1, C), lambda i: (0, 0)),
        ],
        out_specs=[
            pl.BlockSpec((BB, C), lambda i: (i, 0)),
            pl.BlockSpec((BB, C), lambda i: (i, 0)),
        ],
        out_shape=[
            jax.ShapeDtypeStruct((B, C), jnp.float32),
            jax.ShapeDtypeStruct((B, C), jnp.float32),
        ],
    )(adjsum, rT, W1 * scale, b1.reshape(1, F), W2 * scale,
      b2.reshape(1, F), Wlin.reshape(1, F), blin.reshape(1, 1), Whead.T,
      bhead.reshape(1, C))
    return out1, out2


def kernel(real, imag, graph, W1, b1, W2, b2, Wlin, blin, Whead, bhead, layer):
    del imag
    out1, out2 = _run(real, graph, W1, b1, W2, b2, Wlin, blin, Whead, bhead)
    return jnp.where(layer > 1, out2, out1)


# speedup vs baseline: 3.6119x; 3.3857x over previous
import functools

import jax
import jax.numpy as jnp
from jax.experimental import pallas as pl
from jax.experimental.pallas import tpu as pltpu

B = 4096
K = 8
N = 30
IN_C = 128
F = 64
C = 5
BB = 128


RR = 128
NNP = 1024


def _adj_kernel(g_ref, out_ref):
    g = g_ref[...].reshape(RR, K, B)
    s = jnp.sum(g, axis=1)
    out_ref[...] = s.T


def _net_kernel(adj_ref, real_ref, w1_ref, b1_ref, w2_ref, b2_ref,
                wlin_ref, blin_ref, wheadt_ref, bhead_ref,
                out1_ref, out2_ref):
    adj = adj_ref[:, : N * N].reshape(BB, N, N)

    r = real_ref[...]
    h = jax.lax.dot_general(
        r, w1_ref[...],
        dimension_numbers=(((2,), (0,)), ((), ())),
        preferred_element_type=jnp.float32)
    h = jnp.transpose(h, (1, 0, 2))

    x = jax.lax.dot_general(
        adj, h,
        dimension_numbers=(((2,), (1,)), ((0,), (0,))),
        preferred_element_type=jnp.float32)
    x = jnp.maximum(x + b1_ref[...].reshape(1, 1, F), 0.0)

    h2 = jax.lax.dot_general(
        x, w2_ref[...],
        dimension_numbers=(((2,), (0,)), ((), ())),
        preferred_element_type=jnp.float32)
    x2 = jax.lax.dot_general(
        adj, h2,
        dimension_numbers=(((2,), (1,)), ((0,), (0,))),
        preferred_element_type=jnp.float32)
    x2 = jnp.maximum(x2 + b2_ref[...].reshape(1, 1, F), 0.0)

    wlin = wlin_ref[...].reshape(1, 1, F)
    blin = blin_ref[0, 0]
    wheadt = wheadt_ref[...]
    bhead = bhead_ref[...]

    def head(xk, out_ref):
        xl = jnp.sum(xk * wlin, axis=2) + blin
        xr = jnp.maximum(xl, 0.0)
        out = jax.lax.dot_general(
            xr, wheadt,
            dimension_numbers=(((1,), (0,)), ((), ())),
            preferred_element_type=jnp.float32)
        out_ref[...] = out + bhead

    head(x, out1_ref)
    head(x2, out2_ref)


@functools.partial(jax.jit, static_argnames=())
def _run(real, graph, W1, b1, W2, b2, Wlin, blin, Whead, bhead):
    gflat = jnp.transpose(graph, (2, 3, 1, 0)).reshape(N * N * K, B)
    rT = jnp.transpose(real, (1, 0, 2))

    adjsum = pl.pallas_call(
        _adj_kernel,
        grid=(NNP // RR,),
        in_specs=[pl.BlockSpec((RR * K, B), lambda r: (r, 0))],
        out_specs=pl.BlockSpec((B, RR), lambda r: (0, r)),
        out_shape=jax.ShapeDtypeStruct((B, NNP), jnp.float32),
    )(gflat)

    scale = jnp.float32(1.0 / K)
    out1, out2 = pl.pallas_call(
        _net_kernel,
        grid=(B // BB,),
        in_specs=[
            pl.BlockSpec((BB, NNP), lambda i: (i, 0)),
            pl.BlockSpec((N, BB, IN_C), lambda i: (0, i, 0)),
            pl.BlockSpec((IN_C, F), lambda i: (0, 0)),
            pl.BlockSpec((1, F), lambda i: (0, 0)),
            pl.BlockSpec((F, F), lambda i: (0, 0)),
            pl.BlockSpec((1, F), lambda i: (0, 0)),
            pl.BlockSpec((1, F), lambda i: (0, 0)),
            pl.BlockSpec((1, 1), lambda i: (0, 0)),
            pl.BlockSpec((N, C), lambda i: (0, 0)),
            pl.BlockSpec((1, C), lambda i: (0, 0)),
        ],
        out_specs=[
            pl.BlockSpec((BB, C), lambda i: (i, 0)),
            pl.BlockSpec((BB, C), lambda i: (i, 0)),
        ],
        out_shape=[
            jax.ShapeDtypeStruct((B, C), jnp.float32),
            jax.ShapeDtypeStruct((B, C), jnp.float32),
        ],
    )(adjsum, rT, W1 * scale, b1.reshape(1, F), W2 * scale,
      b2.reshape(1, F), Wlin.reshape(1, F), blin.reshape(1, 1), Whead.T,
      bhead.reshape(1, C))
    return out1, out2


def kernel(real, imag, graph, W1, b1, W2, b2, Wlin, blin, Whead, bhead, layer):
    del imag
    out1, out2 = _run(real, graph, W1, b1, W2, b2, Wlin, blin, Whead, bhead)
    return jnp.where(layer > 1, out2, out1)


# speedup vs baseline: 3.6476x; 1.0099x over previous
import functools

import jax
import jax.numpy as jnp
from jax.experimental import pallas as pl
from jax.experimental.pallas import tpu as pltpu

B = 4096
K = 8
N = 30
IN_C = 128
F = 64
C = 5
BB = 256


RR = 128
NNP = 1024


def _adj_kernel(g_ref, out_ref):
    g = g_ref[...].reshape(RR, K, B)
    s = jnp.sum(g, axis=1)
    out_ref[...] = s.T


def _net_kernel(adj_ref, real_ref, w1_ref, b1_ref, w2_ref, b2_ref,
                wlin_ref, blin_ref, wheadt_ref, bhead_ref,
                out1_ref, out2_ref):
    adj = adj_ref[:, : N * N].reshape(BB, N, N)

    r = real_ref[...]
    h = jax.lax.dot_general(
        r, w1_ref[...],
        dimension_numbers=(((2,), (0,)), ((), ())),
        preferred_element_type=jnp.float32)
    h = jnp.transpose(h, (1, 0, 2))

    x = jax.lax.dot_general(
        adj, h,
        dimension_numbers=(((2,), (1,)), ((0,), (0,))),
        preferred_element_type=jnp.float32)
    x = jnp.maximum(x + b1_ref[...].reshape(1, 1, F), 0.0)

    h2 = jax.lax.dot_general(
        x, w2_ref[...],
        dimension_numbers=(((2,), (0,)), ((), ())),
        preferred_element_type=jnp.float32)
    x2 = jax.lax.dot_general(
        adj, h2,
        dimension_numbers=(((2,), (1,)), ((0,), (0,))),
        preferred_element_type=jnp.float32)
    x2 = jnp.maximum(x2 + b2_ref[...].reshape(1, 1, F), 0.0)

    wlin = wlin_ref[...].reshape(1, 1, F)
    blin = blin_ref[0, 0]
    wheadt = wheadt_ref[...]
    bhead = bhead_ref[...]

    def head(xk, out_ref):
        xl = jnp.sum(xk * wlin, axis=2) + blin
        xr = jnp.maximum(xl, 0.0)
        out = jax.lax.dot_general(
            xr, wheadt,
            dimension_numbers=(((1,), (0,)), ((), ())),
            preferred_element_type=jnp.float32)
        out_ref[...] = out + bhead

    head(x, out1_ref)
    head(x2, out2_ref)


@functools.partial(jax.jit, static_argnames=())
def _run(real, graph, W1, b1, W2, b2, Wlin, blin, Whead, bhead):
    gflat = jnp.transpose(graph, (2, 3, 1, 0)).reshape(N * N * K, B)
    rT = jnp.transpose(real, (1, 0, 2))

    adjsum = pl.pallas_call(
        _adj_kernel,
        grid=(NNP // RR,),
        in_specs=[pl.BlockSpec((RR * K, B), lambda r: (r, 0))],
        out_specs=pl.BlockSpec((B, RR), lambda r: (0, r)),
        out_shape=jax.ShapeDtypeStruct((B, NNP), jnp.float32),
    )(gflat)

    scale = jnp.float32(1.0 / K)
    out1, out2 = pl.pallas_call(
        _net_kernel,
        grid=(B // BB,),
        in_specs=[
            pl.BlockSpec((BB, NNP), lambda i: (i, 0)),
            pl.BlockSpec((N, BB, IN_C), lambda i: (0, i, 0)),
            pl.BlockSpec((IN_C, F), lambda i: (0, 0)),
            pl.BlockSpec((1, F), lambda i: (0, 0)),
            pl.BlockSpec((F, F), lambda i: (0, 0)),
            pl.BlockSpec((1, F), lambda i: (0, 0)),
            pl.BlockSpec((1, F), lambda i: (0, 0)),
            pl.BlockSpec((1, 1), lambda i: (0, 0)),
            pl.BlockSpec((N, C), lambda i: (0, 0)),
            pl.BlockSpec((1, C), lambda i: (0, 0)),
        ],
        out_specs=[
            pl.BlockSpec((BB, C), lambda i: (i, 0)),
            pl.BlockSpec((BB, C), lambda i: (i, 0)),
        ],
        out_shape=[
            jax.ShapeDtypeStruct((B, C), jnp.float32),
            jax.ShapeDtypeStruct((B, C), jnp.float32),
        ],
    )(adjsum, rT, W1 * scale, b1.reshape(1, F), W2 * scale,
      b2.reshape(1, F), Wlin.reshape(1, F), blin.reshape(1, 1), Whead.T,
      bhead.reshape(1, C))
    return out1, out2


def kernel(real, imag, graph, W1, b1, W2, b2, Wlin, blin, Whead, bhead, layer):
    del imag
    out1, out2 = _run(real, graph, W1, b1, W2, b2, Wlin, blin, Whead, bhead)
    return jnp.where(layer > 1, out2, out1)
